# Initial kernel scaffold; baseline (speedup 1.0000x reference)
import jax
import jax.numpy as jnp
from jax.experimental import pallas as pl


def kernel(a, neg_gamma, value):
    raise NotImplementedError("write your pallas kernel here")


# speedup vs baseline: 11.9817x; 11.9817x over previous
import functools

import jax
import jax.numpy as jnp
from jax import lax
from jax.experimental import pallas as pl
from jax.experimental.pallas import tpu as pltpu
from jax.experimental.pallas import tpu_sc as plsc

K_BINS = 50
N = 1_000_000
NW = 32
SPAN = 31_360
NCHUNK = 5
CHUNK = 6_272
ROWS = CHUNK // 128
LAST_BASE = N - SPAN

XY_LO, XY_HI = -4.5, 4.5
Z_LO, Z_HI = -11.0, 11.0
XY_INV = K_BINS / (XY_HI - XY_LO)
Z_INV = K_BINS / (Z_HI - Z_LO)


def _bucketize(v, tbl_ref, lo, inv):
    t = jnp.clip((v - lo) * inv, -1.0, 51.0)
    c0 = t.astype(jnp.int32)
    j = jnp.clip(c0 + 1, 0, 52)
    blo = plsc.load_gather(tbl_ref, [j])
    bhi = plsc.load_gather(tbl_ref, [j + 1])
    c = jnp.where(bhi < v, c0 + 1, jnp.where(blo >= v, c0 - 1, c0))
    return jnp.where(c < 0, 49, jnp.minimum(c, 49))


def _sc_body(a_hbm, z_hbm, tx_hbm, tz_hbm, val_hbm, out_hbm,
             a_v, z_v, idx_v, rows_v, w_v, tx_v, tz_v, sem):
    wid = lax.axis_index("s") * 2 + lax.axis_index("c")
    base = jnp.minimum(wid * 31_250 // 8 * 8, LAST_BASE)

    pltpu.sync_copy(tx_hbm, tx_v)
    pltpu.sync_copy(tz_hbm, tz_v)

    ev = lax.iota(jnp.int32, 16) * 2
    od = ev + 1

    def do_chunk(k, _):
        cbase = base + k * CHUNK
        pltpu.sync_copy(a_hbm.at[pl.ds(2 * cbase, 2 * CHUNK)], a_v)
        pltpu.sync_copy(z_hbm.at[pl.ds(cbase, CHUNK)], z_v)

        def row(r, _):
            for l in range(8):
                v0 = a_v[pl.ds(256 * r + 32 * l, 16)]
                v1 = a_v[pl.ds(256 * r + 32 * l + 16, 16)]
                i0 = _bucketize(v0, tx_v, XY_LO, XY_INV)
                i1 = _bucketize(v1, tx_v, XY_LO, XY_INV)
                w_v[pl.ds(32 * l, 16)] = i0
                w_v[pl.ds(32 * l + 16, 16)] = i1
                xi = plsc.load_gather(w_v, [ev + 32 * l])
                yi = plsc.load_gather(w_v, [od + 32 * l])
                zi = _bucketize(z_v[pl.ds(128 * r + 16 * l, 16)],
                                tz_v, Z_LO, Z_INV)
                idx_v[r, pl.ds(16 * l, 16)] = (xi * 50 + yi) * 50 + zi
            return 0

        lax.fori_loop(0, ROWS, row, 0)

        copies = [
            pltpu.async_copy(val_hbm.at[idx_v.at[j]],
                             rows_v.at[pl.ds(128 * j, 128)], sem)
            for j in range(ROWS)
        ]
        for c in copies:
            c.wait()
        pltpu.sync_copy(rows_v, out_hbm.at[pl.ds(cbase, CHUNK)])
        return 0

    lax.fori_loop(0, NCHUNK, do_chunk, 0)


@jax.jit
def kernel(a, neg_gamma, value):
    inf = jnp.float32(jnp.inf)
    tx = jnp.concatenate([jnp.array([-inf]),
                          jnp.linspace(XY_LO, XY_HI, K_BINS + 1),
                          jnp.full((12,), inf)])
    tz = jnp.concatenate([jnp.array([-inf]),
                          jnp.linspace(Z_LO, Z_HI, K_BINS + 1),
                          jnp.full((12,), inf)])
    run = pl.kernel(
        _sc_body,
        out_type=jax.ShapeDtypeStruct((N, 16), jnp.float32),
        mesh=plsc.VectorSubcoreMesh(core_axis_name="c", subcore_axis_name="s"),
        compiler_params=pltpu.CompilerParams(needs_layout_passes=False,
                                             use_tc_tiling_on_sc=False),
        scratch_types=[
            pltpu.VMEM((2 * CHUNK,), jnp.float32),
            pltpu.VMEM((CHUNK,), jnp.float32),
            pltpu.VMEM((ROWS, 128), jnp.int32),
            pltpu.VMEM((CHUNK, 16), jnp.float32),
            pltpu.VMEM((256,), jnp.int32),
            pltpu.VMEM((64,), jnp.float32),
            pltpu.VMEM((64,), jnp.float32),
            pltpu.SemaphoreType.DMA,
        ],
    )
    val16 = jnp.pad(value.reshape(-1, 2), ((0, 0), (0, 14)))
    out = run(a.reshape(-1), neg_gamma, tx, tz, val16)
    return out[:, :2][None]


# speedup vs baseline: 12.4559x; 1.0396x over previous
import functools

import jax
import jax.numpy as jnp
from jax import lax
from jax.experimental import pallas as pl
from jax.experimental.pallas import tpu as pltpu
from jax.experimental.pallas import tpu_sc as plsc

K_BINS = 50
N = 1_000_000
NW = 32
SPAN = 31_360
NCHUNK = 5
CHUNK = 6_272
ROWS = CHUNK // 128
LAST_BASE = N - SPAN

XY_LO, XY_HI = -4.5, 4.5
Z_LO, Z_HI = -11.0, 11.0
XY_INV = K_BINS / (XY_HI - XY_LO)
Z_INV = K_BINS / (Z_HI - Z_LO)


def _bucketize(v, tbl_ref, lo, inv):
    t = jnp.clip((v - lo) * inv, -1.0, 51.0)
    c0 = t.astype(jnp.int32)
    j = jnp.clip(c0 + 1, 0, 52)
    blo = plsc.load_gather(tbl_ref, [j])
    bhi = plsc.load_gather(tbl_ref, [j + 1])
    c = jnp.where(bhi < v, c0 + 1, jnp.where(blo >= v, c0 - 1, c0))
    return jnp.where(c < 0, 49, jnp.minimum(c, 49))


def _sc_body(a_hbm, z_hbm, tx_hbm, tz_hbm, val_hbm, out_hbm,
             a_v, z_v, idx_v, rows_v, w_v, tx_v, tz_v, sem):
    wid = lax.axis_index("s") * 2 + lax.axis_index("c")
    base = jnp.minimum(wid * 31_250 // 8 * 8, LAST_BASE)

    pltpu.sync_copy(tx_hbm, tx_v)
    pltpu.sync_copy(tz_hbm, tz_v)

    lanes = lax.iota(jnp.int32, 16)
    ev = lanes * 2
    od = ev + 1
    qhalf = lanes >> 1
    par = lanes & 1

    def do_chunk(k, _):
        cbase = base + k * CHUNK
        pltpu.sync_copy(a_hbm.at[pl.ds(2 * cbase, 2 * CHUNK)], a_v)
        pltpu.sync_copy(z_hbm.at[pl.ds(cbase, CHUNK)], z_v)

        def row(r, _):
            for l in range(8):
                v0 = a_v[pl.ds(256 * r + 32 * l, 16)]
                v1 = a_v[pl.ds(256 * r + 32 * l + 16, 16)]
                i0 = _bucketize(v0, tx_v, XY_LO, XY_INV)
                i1 = _bucketize(v1, tx_v, XY_LO, XY_INV)
                w_v[pl.ds(32 * l, 16)] = i0
                w_v[pl.ds(32 * l + 16, 16)] = i1
                xi = plsc.load_gather(w_v, [ev + 32 * l])
                yi = plsc.load_gather(w_v, [od + 32 * l])
                zi = _bucketize(z_v[pl.ds(128 * r + 16 * l, 16)],
                                tz_v, Z_LO, Z_INV)
                idx_v[r, pl.ds(16 * l, 16)] = (xi * 50 + yi) * 50 + zi
            return 0

        lax.fori_loop(0, ROWS, row, 0)

        copies = [
            pltpu.async_copy(val_hbm.at[idx_v.at[j]],
                             rows_v.at[pl.ds(128 * j, 128)], sem)
            for j in range(ROWS)
        ]
        for c in copies:
            c.wait()

        def crow(r, _):
            for l in range(16):
                pv = qhalf + (128 * r + 8 * l)
                a_v[pl.ds(256 * r + 16 * l, 16)] = plsc.load_gather(
                    rows_v, [pv, par])
            return 0

        lax.fori_loop(0, ROWS, crow, 0)
        pltpu.sync_copy(a_v, out_hbm.at[pl.ds(2 * cbase, 2 * CHUNK)])
        return 0

    lax.fori_loop(0, NCHUNK, do_chunk, 0)


@jax.jit
def kernel(a, neg_gamma, value):
    inf = jnp.float32(jnp.inf)
    tx = jnp.concatenate([jnp.array([-inf]),
                          jnp.linspace(XY_LO, XY_HI, K_BINS + 1),
                          jnp.full((12,), inf)])
    tz = jnp.concatenate([jnp.array([-inf]),
                          jnp.linspace(Z_LO, Z_HI, K_BINS + 1),
                          jnp.full((12,), inf)])
    run = pl.kernel(
        _sc_body,
        out_type=jax.ShapeDtypeStruct((2 * N,), jnp.float32),
        mesh=plsc.VectorSubcoreMesh(core_axis_name="c", subcore_axis_name="s"),
        compiler_params=pltpu.CompilerParams(needs_layout_passes=False,
                                             use_tc_tiling_on_sc=False),
        scratch_types=[
            pltpu.VMEM((2 * CHUNK,), jnp.float32),
            pltpu.VMEM((CHUNK,), jnp.float32),
            pltpu.VMEM((ROWS, 128), jnp.int32),
            pltpu.VMEM((CHUNK, 16), jnp.float32),
            pltpu.VMEM((256,), jnp.int32),
            pltpu.VMEM((64,), jnp.float32),
            pltpu.VMEM((64,), jnp.float32),
            pltpu.SemaphoreType.DMA,
        ],
    )
    val16 = jnp.pad(value.reshape(-1, 2), ((0, 0), (0, 14)))
    out = run(a.reshape(-1), neg_gamma, tx, tz, val16)
    return out.reshape(1, N, 2)


# speedup vs baseline: 83.8843x; 6.7345x over previous
import jax
import jax.numpy as jnp
from jax import lax
from jax.experimental import pallas as pl
from jax.experimental.pallas import tpu as pltpu
from jax.experimental.pallas import tpu_sc as plsc

K_BINS = 50
N = 1_000_000
SPAN = 31_360
NCHUNK = 5
CHUNK = 6_272
ROWS = CHUNK // 128
LAST_BASE = N - SPAN

XY_LO, XY_HI = -4.5, 4.5
Z_LO, Z_HI = -11.0, 11.0
XY_INV = K_BINS / (XY_HI - XY_LO)
Z_INV = K_BINS / (Z_HI - Z_LO)


def _bucketize(v, tbl_ref, lo, inv):
    t = jnp.clip((v - lo) * inv, -1.0, 51.0)
    c0 = t.astype(jnp.int32)
    j = jnp.clip(c0 + 1, 0, 52)
    blo = plsc.load_gather(tbl_ref, [j])
    bhi = plsc.load_gather(tbl_ref, [j + 1])
    c = jnp.where(bhi < v, c0 + 1, jnp.where(blo >= v, c0 - 1, c0))
    return jnp.where(c < 0, 49, jnp.minimum(c, 49))


def _sc_body(x_hbm, y_hbm, z_hbm, tx_hbm, tz_hbm, val_hbm, out_hbm,
             x_v, y_v, z_v, idx_v, rows_v, tx_v, tz_v, sem):
    wid = lax.axis_index("s") * 2 + lax.axis_index("c")
    base = jnp.minimum(wid * 31_250 // 8 * 8, LAST_BASE)

    pltpu.sync_copy(tx_hbm, tx_v)
    pltpu.sync_copy(tz_hbm, tz_v)

    lanes = lax.iota(jnp.int32, 16)
    zero16 = lanes * 0
    one16 = zero16 + 1

    def do_chunk(k, _):
        cbase = base + k * CHUNK
        pltpu.sync_copy(x_hbm.at[pl.ds(cbase, CHUNK)], x_v)
        pltpu.sync_copy(y_hbm.at[pl.ds(cbase, CHUNK)], y_v)
        pltpu.sync_copy(z_hbm.at[pl.ds(cbase, CHUNK)], z_v)

        def row(r, _):
            for l in range(8):
                o = 128 * r + 16 * l
                xi = _bucketize(x_v[pl.ds(o, 16)], tx_v, XY_LO, XY_INV)
                yi = _bucketize(y_v[pl.ds(o, 16)], tx_v, XY_LO, XY_INV)
                zi = _bucketize(z_v[pl.ds(o, 16)], tz_v, Z_LO, Z_INV)
                idx_v[r, pl.ds(16 * l, 16)] = (xi * 50 + yi) * 50 + zi
            return 0

        lax.fori_loop(0, ROWS, row, 0)

        copies = [
            pltpu.async_copy(val_hbm.at[idx_v.at[j]],
                             rows_v.at[pl.ds(128 * j, 128)], sem)
            for j in range(ROWS)
        ]
        for c in copies:
            c.wait()

        def crow(r, _):
            for l in range(8):
                o = 128 * r + 16 * l
                pv = lanes + o
                x_v[pl.ds(o, 16)] = plsc.load_gather(rows_v, [pv, zero16])
                y_v[pl.ds(o, 16)] = plsc.load_gather(rows_v, [pv, one16])
            return 0

        lax.fori_loop(0, ROWS, crow, 0)
        pltpu.sync_copy(x_v, out_hbm.at[0, pl.ds(cbase, CHUNK)])
        pltpu.sync_copy(y_v, out_hbm.at[1, pl.ds(cbase, CHUNK)])
        return 0

    lax.fori_loop(0, NCHUNK, do_chunk, 0)


@jax.jit
def kernel(a, neg_gamma, value):
    inf = jnp.float32(jnp.inf)
    tx = jnp.concatenate([jnp.array([-inf]),
                          jnp.linspace(XY_LO, XY_HI, K_BINS + 1),
                          jnp.full((12,), inf)])
    tz = jnp.concatenate([jnp.array([-inf]),
                          jnp.linspace(Z_LO, Z_HI, K_BINS + 1),
                          jnp.full((12,), inf)])
    run = pl.kernel(
        _sc_body,
        out_type=jax.ShapeDtypeStruct((2, N), jnp.float32),
        mesh=plsc.VectorSubcoreMesh(core_axis_name="c", subcore_axis_name="s"),
        compiler_params=pltpu.CompilerParams(needs_layout_passes=False,
                                             use_tc_tiling_on_sc=False),
        scratch_types=[
            pltpu.VMEM((CHUNK,), jnp.float32),
            pltpu.VMEM((CHUNK,), jnp.float32),
            pltpu.VMEM((CHUNK,), jnp.float32),
            pltpu.VMEM((ROWS, 128), jnp.int32),
            pltpu.VMEM((CHUNK, 16), jnp.float32),
            pltpu.VMEM((64,), jnp.float32),
            pltpu.VMEM((64,), jnp.float32),
            pltpu.SemaphoreType.DMA,
        ],
    )
    val16 = jnp.pad(value.reshape(-1, 2), ((0, 0), (0, 14)))
    out = run(a[:, 0], a[:, 1], neg_gamma, tx, tz, val16)
    return out.T[None]
